# Initial kernel scaffold; baseline (speedup 1.0000x reference)
import jax
import jax.numpy as jnp
from jax.experimental import pallas as pl


def kernel(x, edge_index, edge_attr, W1, b1, W2, b2, Wfc, bfc):
    raise NotImplementedError("write your pallas kernel here")


# speedup vs baseline: 3.5336x; 3.5336x over previous
import functools

import jax
import jax.numpy as jnp
from jax import lax
from jax.experimental import pallas as pl
from jax.experimental.pallas import tpu as pltpu
from jax.experimental.pallas import tpu_sc as plsc

N = 10000
NP = 10240
H = 128
LANES = 16
NC, NS = 2, 16
NW = NC * NS
CHUNK = 64
RPT = NP // NS
RQ = RPT // 5

_F32 = jnp.float32
_I32 = jnp.int32


def _mesh():
    return plsc.VectorSubcoreMesh(
        core_axis_name="c", subcore_axis_name="s",
        num_cores=NC, num_subcores=NS)


def _worker_id():
    return lax.axis_index("s") * NC + lax.axis_index("c")


KB = 8


def _make_msg_kernel(cpw, nchunk):
    @functools.partial(
        pl.kernel,
        out_type=jax.ShapeDtypeStruct((NC, NP, H), _F32),
        mesh=_mesh(),
        scratch_types=[
            pltpu.VMEM((KB, CHUNK), _I32),
            pltpu.VMEM((KB, CHUNK), _I32),
            pltpu.VMEM((CHUNK, H), _F32),
            pltpu.VMEM((CHUNK, H), _F32),
            pltpu.VMEM((CHUNK, LANES), _F32),
            pltpu.VMEM((CHUNK, LANES), _F32),
            pltpu.VMEM_SHARED((NP, H), _F32),
            pltpu.SemaphoreType.DMA,
            pltpu.SemaphoreType.DMA,
            pltpu.SemaphoreType.DMA,
            pltpu.SemaphoreType.DMA,
        ],
    )
    def msg_kernel(g2, row2d, col2d, w3d, out,
                   ridx, cidx, rows0, rows1, wv0, wv1, acc,
                   g0, g1, m0, m1):
        c = lax.axis_index("c")
        s = lax.axis_index("s")
        wid = _worker_id()

        def zrow(r, _):
            for q in range(H // LANES):
                rows0[r, pl.ds(q * LANES, LANES)] = jnp.zeros((LANES,), _F32)
            return 0

        lax.fori_loop(0, CHUNK, zrow, 0, unroll=4)
        for k in range(RPT // CHUNK):
            pltpu.sync_copy(rows0, acc.at[pl.ds(s * RPT + k * CHUNK, CHUNK)])
        plsc.subcore_barrier()

        def scale(rows, wv):
            def ebody(e, _):
                wsplat = wv[e, :]
                for q in range(H // LANES):
                    sl = pl.ds(q * LANES, LANES)
                    rows[e, sl] = rows[e, sl] * wsplat
                return 0

            lax.fori_loop(0, CHUNK, ebody, 0, unroll=4)

        def blk(bi, _):
            base = wid * cpw + bi * KB
            pltpu.sync_copy(row2d.at[pl.ds(base, KB)], ridx)
            pltpu.sync_copy(col2d.at[pl.ds(base, KB)], cidx)

            def body(jj, _):
                j0 = 2 * jj
                j1 = j0 + 1
                d0 = pltpu.async_copy(g2.at[ridx.at[j0]], rows0, g0)
                n0 = pltpu.async_copy(w3d.at[base + j0], wv0, m0)
                d1 = pltpu.async_copy(g2.at[ridx.at[j1]], rows1, g1)
                n1 = pltpu.async_copy(w3d.at[base + j1], wv1, m1)
                d0.wait()
                n0.wait()
                scale(rows0, wv0)
                pltpu.sync_copy(rows0, acc.at[cidx.at[j0]], add=True)
                d1.wait()
                n1.wait()
                scale(rows1, wv1)
                pltpu.sync_copy(rows1, acc.at[cidx.at[j1]], add=True)
                return 0

            lax.fori_loop(0, KB // 2, body, 0)
            return 0

        lax.fori_loop(0, cpw // KB, blk, 0)
        plsc.subcore_barrier()
        pltpu.sync_copy(acc.at[pl.ds(s * RPT, RPT)],
                        out.at[c, pl.ds(s * RPT, RPT)])

    return msg_kernel


_BR = 1000


def _dis_body(d0_ref, d1_ref, dis_ref):
    deg = d0_ref[...] + d1_ref[...] + 2.0
    dis_ref[...] = jnp.where(deg > 0.0, lax.rsqrt(deg), 0.0)


def _dis_kernel(d0, d1):
    return pl.pallas_call(
        _dis_body,
        out_shape=jax.ShapeDtypeStruct(d0.shape, _F32),
    )(d0, d1)


def _mm_body(x_ref, w_ref, dis_ref, o_ref):
    o_ref[...] = dis_ref[:, 0:1] * jnp.dot(x_ref[...], w_ref[...],
                                           preferred_element_type=_F32)


def _mm(x, w, dis16):
    n, k = x.shape
    return pl.pallas_call(
        _mm_body,
        grid=(n // _BR,),
        in_specs=[pl.BlockSpec((_BR, k), lambda i: (i, 0)),
                  pl.BlockSpec((k, w.shape[1]), lambda i: (0, 0)),
                  pl.BlockSpec((_BR, H), lambda i: (i, 0))],
        out_specs=pl.BlockSpec((_BR, w.shape[1]), lambda i: (i, 0)),
        out_shape=jax.ShapeDtypeStruct((n, w.shape[1]), _F32),
    )(x, w, dis16)


def _epi_mm_body(acc_ref0, acc_ref1, g_ref, dis_ref, b_ref, w_ref, o_ref):
    dis = dis_ref[:, 0:1]
    act = dis * (acc_ref0[0] + acc_ref1[0] + 2.0 * g_ref[...]) + b_ref[...]
    act = jnp.maximum(act, 0.0)
    o_ref[...] = dis * jnp.dot(act, w_ref[...], preferred_element_type=_F32)


def _epi_mm(acc, g, dis16, b, w):
    return pl.pallas_call(
        _epi_mm_body,
        grid=(N // _BR,),
        in_specs=[pl.BlockSpec((1, _BR, H), lambda i: (0, i, 0)),
                  pl.BlockSpec((1, _BR, H), lambda i: (1, i, 0)),
                  pl.BlockSpec((_BR, H), lambda i: (i, 0)),
                  pl.BlockSpec((_BR, H), lambda i: (i, 0)),
                  pl.BlockSpec((1, H), lambda i: (0, 0)),
                  pl.BlockSpec((H, H), lambda i: (0, 0))],
        out_specs=pl.BlockSpec((_BR, H), lambda i: (i, 0)),
        out_shape=jax.ShapeDtypeStruct((N, H), _F32),
    )(acc, acc, g, dis16, b, w)


def _final_body(acc_ref0, acc_ref1, g_ref, dis_ref, b_ref, wfc_ref, bfc_ref,
                o_ref):
    dis = dis_ref[:, 0:1]
    act = dis * (acc_ref0[0] + acc_ref1[0] + 2.0 * g_ref[...]) + b_ref[...]
    act = jnp.maximum(act, 0.0)
    logits = jnp.dot(act, wfc_ref[...], preferred_element_type=_F32)
    logits = logits + bfc_ref[...]
    m = jnp.max(logits, axis=-1, keepdims=True)
    e = jnp.exp(logits - m)
    o_ref[...] = e / jnp.sum(e, axis=-1, keepdims=True)


def _final(acc, g, dis16, b, wfcp, bfcp):
    return pl.pallas_call(
        _final_body,
        grid=(N // _BR,),
        in_specs=[pl.BlockSpec((1, _BR, H), lambda i: (0, i, 0)),
                  pl.BlockSpec((1, _BR, H), lambda i: (1, i, 0)),
                  pl.BlockSpec((_BR, H), lambda i: (i, 0)),
                  pl.BlockSpec((_BR, H), lambda i: (i, 0)),
                  pl.BlockSpec((1, H), lambda i: (0, 0)),
                  pl.BlockSpec((H, H), lambda i: (0, 0)),
                  pl.BlockSpec((1, H), lambda i: (0, 0))],
        out_specs=pl.BlockSpec((_BR, H), lambda i: (i, 0)),
        out_shape=jax.ShapeDtypeStruct((N, H), _F32),
    )(acc, acc, g, dis16, b, wfcp, bfcp)


def kernel(x, edge_index, edge_attr, W1, b1, W2, b2, Wfc, bfc):
    E = edge_index.shape[1]
    cpw = -(-E // (CHUNK * NW))
    cpw = -(-cpw // KB) * KB
    epad = cpw * CHUNK * NW
    nchunk = epad // CHUNK

    ei = edge_index.astype(_I32)
    row = jnp.pad(ei[0], (0, epad - E))
    col = jnp.pad(ei[1], (0, epad - E))
    w = jnp.pad(edge_attr.astype(_F32), (0, epad - E))

    row2d = row.reshape(nchunk, CHUNK)
    col2d = col.reshape(nchunk, CHUNK)
    w3d = jnp.broadcast_to(w[:, None], (epad, LANES)).reshape(
        nchunk, CHUNK, LANES)

    msg = _make_msg_kernel(cpw, nchunk)

    deg2 = msg(jnp.ones((N, H), _F32), row2d, col2d, w3d)
    dis16 = _dis_kernel(deg2[0], deg2[1])

    b1r = b1.reshape(1, H)
    b2r = b2.reshape(1, H)
    wfcp = jnp.zeros((H, H), _F32).at[:, :Wfc.shape[1]].set(Wfc)
    bfcp = jnp.full((1, H), -1e30, _F32).at[0, :bfc.shape[0]].set(bfc)

    g = _mm(x, W1, dis16)
    acc = msg(g, row2d, col2d, w3d)
    g = _epi_mm(acc, g, dis16, b1r, W2)
    for _ in range(5):
        acc = msg(g, row2d, col2d, w3d)
        g = _epi_mm(acc, g, dis16, b2r, W2)
    acc = msg(g, row2d, col2d, w3d)
    probs = _final(acc, g, dis16, b2r, wfcp, bfcp)
    return probs[:, :Wfc.shape[1]]


# speedup vs baseline: 3.5357x; 1.0006x over previous
import functools

import jax
import jax.numpy as jnp
from jax import lax
from jax.experimental import pallas as pl
from jax.experimental.pallas import tpu as pltpu
from jax.experimental.pallas import tpu_sc as plsc

N = 10000
NP = 10240
H = 128
LANES = 16
NC, NS = 2, 16
NW = NC * NS
CHUNK = 64
RPT = NP // NS
RQ = RPT // 5

_F32 = jnp.float32
_I32 = jnp.int32


def _mesh():
    return plsc.VectorSubcoreMesh(
        core_axis_name="c", subcore_axis_name="s",
        num_cores=NC, num_subcores=NS)


def _worker_id():
    return lax.axis_index("s") * NC + lax.axis_index("c")


KB = 8


def _make_msg_kernel(cpw, nchunk):
    @functools.partial(
        pl.kernel,
        out_type=jax.ShapeDtypeStruct((NC, NP, H), _F32),
        mesh=_mesh(),
        scratch_types=[
            pltpu.VMEM((KB, CHUNK), _I32),
            pltpu.VMEM((KB, CHUNK), _I32),
            pltpu.VMEM((CHUNK, H), _F32),
            pltpu.VMEM((CHUNK, H), _F32),
            pltpu.VMEM((CHUNK, LANES), _F32),
            pltpu.VMEM((CHUNK, LANES), _F32),
            pltpu.VMEM_SHARED((NP, H), _F32),
            pltpu.SemaphoreType.DMA,
            pltpu.SemaphoreType.DMA,
            pltpu.SemaphoreType.DMA,
            pltpu.SemaphoreType.DMA,
        ],
    )
    def msg_kernel(g2, row2d, col2d, w3d, out,
                   ridx, cidx, rows0, rows1, wv0, wv1, acc,
                   g0, g1, m0, m1):
        c = lax.axis_index("c")
        s = lax.axis_index("s")
        wid = _worker_id()

        def zrow(r, _):
            for q in range(H // LANES):
                rows0[r, pl.ds(q * LANES, LANES)] = jnp.zeros((LANES,), _F32)
            return 0

        lax.fori_loop(0, CHUNK, zrow, 0, unroll=4)
        for k in range(RPT // CHUNK):
            pltpu.sync_copy(rows0, acc.at[pl.ds(s * RPT + k * CHUNK, CHUNK)])
        plsc.subcore_barrier()

        def scale(rows, wv):
            def ebody(e, _):
                wsplat = wv[e, :]
                for q in range(H // LANES):
                    sl = pl.ds(q * LANES, LANES)
                    rows[e, sl] = rows[e, sl] * wsplat
                return 0

            lax.fori_loop(0, CHUNK, ebody, 0, unroll=4)

        def blk(bi, _):
            base = wid * cpw + bi * KB
            pltpu.sync_copy(row2d.at[pl.ds(base, KB)], ridx)
            pltpu.sync_copy(col2d.at[pl.ds(base, KB)], cidx)

            def body(jj, _):
                j0 = 2 * jj
                j1 = j0 + 1
                d0 = pltpu.async_copy(g2.at[ridx.at[j0]], rows0, g0)
                n0 = pltpu.async_copy(w3d.at[base + j0], wv0, m0)
                d1 = pltpu.async_copy(g2.at[ridx.at[j1]], rows1, g1)
                n1 = pltpu.async_copy(w3d.at[base + j1], wv1, m1)
                d0.wait()
                n0.wait()
                scale(rows0, wv0)
                pltpu.sync_copy(rows0, acc.at[cidx.at[j0]], add=True)
                d1.wait()
                n1.wait()
                scale(rows1, wv1)
                pltpu.sync_copy(rows1, acc.at[cidx.at[j1]], add=True)
                return 0

            lax.fori_loop(0, KB // 2, body, 0)
            return 0

        lax.fori_loop(0, cpw // KB, blk, 0)
        plsc.subcore_barrier()
        pltpu.sync_copy(acc.at[pl.ds(s * RPT, RPT)],
                        out.at[c, pl.ds(s * RPT, RPT)])

    return msg_kernel


_BR = 1000


def _dis_body(d0_ref, d1_ref, dis_ref):
    deg = d0_ref[...] + d1_ref[...] + 2.0
    dis_ref[...] = jnp.where(deg > 0.0, lax.rsqrt(deg), 0.0)


def _dis_kernel(d0, d1):
    return pl.pallas_call(
        _dis_body,
        out_shape=jax.ShapeDtypeStruct(d0.shape, _F32),
    )(d0, d1)


def _mm_body(x_ref, w_ref, dis_ref, o_ref):
    o_ref[...] = dis_ref[:, 0:1] * jnp.dot(x_ref[...], w_ref[...],
                                           preferred_element_type=_F32)


def _mm(x, w, dis16):
    n, k = x.shape
    return pl.pallas_call(
        _mm_body,
        grid=(n // _BR,),
        in_specs=[pl.BlockSpec((_BR, k), lambda i: (i, 0)),
                  pl.BlockSpec((k, w.shape[1]), lambda i: (0, 0)),
                  pl.BlockSpec((_BR, H), lambda i: (i, 0))],
        out_specs=pl.BlockSpec((_BR, w.shape[1]), lambda i: (i, 0)),
        out_shape=jax.ShapeDtypeStruct((n, w.shape[1]), _F32),
    )(x, w, dis16)


def _epi_mm_body(acc_ref0, acc_ref1, g_ref, dis_ref, b_ref, w_ref, o_ref):
    dis = dis_ref[:, 0:1]
    act = dis * (acc_ref0[0] + acc_ref1[0] + 2.0 * g_ref[...]) + b_ref[...]
    act = jnp.maximum(act, 0.0)
    o_ref[...] = dis * jnp.dot(act, w_ref[...], preferred_element_type=_F32)


def _epi_mm(acc, g, dis16, b, w):
    return pl.pallas_call(
        _epi_mm_body,
        grid=(N // _BR,),
        in_specs=[pl.BlockSpec((1, _BR, H), lambda i: (0, i, 0)),
                  pl.BlockSpec((1, _BR, H), lambda i: (1, i, 0)),
                  pl.BlockSpec((_BR, H), lambda i: (i, 0)),
                  pl.BlockSpec((_BR, H), lambda i: (i, 0)),
                  pl.BlockSpec((1, H), lambda i: (0, 0)),
                  pl.BlockSpec((H, H), lambda i: (0, 0))],
        out_specs=pl.BlockSpec((_BR, H), lambda i: (i, 0)),
        out_shape=jax.ShapeDtypeStruct((N, H), _F32),
    )(acc, acc, g, dis16, b, w)


def _final_body(acc_ref0, acc_ref1, g_ref, dis_ref, b_ref, wfc_ref, bfc_ref,
                o_ref):
    dis = dis_ref[:, 0:1]
    act = dis * (acc_ref0[0] + acc_ref1[0] + 2.0 * g_ref[...]) + b_ref[...]
    act = jnp.maximum(act, 0.0)
    logits = jnp.dot(act, wfc_ref[...], preferred_element_type=_F32)
    logits = logits + bfc_ref[...]
    m = jnp.max(logits, axis=-1, keepdims=True)
    e = jnp.exp(logits - m)
    o_ref[...] = e / jnp.sum(e, axis=-1, keepdims=True)


def _final(acc, g, dis16, b, wfcp, bfcp):
    return pl.pallas_call(
        _final_body,
        grid=(N // _BR,),
        in_specs=[pl.BlockSpec((1, _BR, H), lambda i: (0, i, 0)),
                  pl.BlockSpec((1, _BR, H), lambda i: (1, i, 0)),
                  pl.BlockSpec((_BR, H), lambda i: (i, 0)),
                  pl.BlockSpec((_BR, H), lambda i: (i, 0)),
                  pl.BlockSpec((1, H), lambda i: (0, 0)),
                  pl.BlockSpec((H, H), lambda i: (0, 0)),
                  pl.BlockSpec((1, H), lambda i: (0, 0))],
        out_specs=pl.BlockSpec((_BR, H), lambda i: (i, 0)),
        out_shape=jax.ShapeDtypeStruct((N, H), _F32),
    )(acc, acc, g, dis16, b, wfcp, bfcp)


def kernel(x, edge_index, edge_attr, W1, b1, W2, b2, Wfc, bfc):
    E = edge_index.shape[1]
    cpw = -(-E // (CHUNK * NW))
    cpw = -(-cpw // KB) * KB
    epad = cpw * CHUNK * NW
    nchunk = epad // CHUNK

    ei = edge_index.astype(_I32)
    row = jnp.pad(ei[0], (0, epad - E))
    pad_cols = jnp.arange(epad - E, dtype=_I32) % NP
    col = jnp.concatenate([ei[1], pad_cols])
    w = jnp.pad(edge_attr.astype(_F32), (0, epad - E))

    row2d = row.reshape(nchunk, CHUNK)
    col2d = col.reshape(nchunk, CHUNK)
    w3d = jnp.broadcast_to(w[:, None], (epad, LANES)).reshape(
        nchunk, CHUNK, LANES)

    msg = _make_msg_kernel(cpw, nchunk)

    deg2 = msg(jnp.ones((N, H), _F32), row2d, col2d, w3d)
    dis16 = _dis_kernel(deg2[0], deg2[1])

    b1r = b1.reshape(1, H)
    b2r = b2.reshape(1, H)
    wfcp = jnp.zeros((H, H), _F32).at[:, :Wfc.shape[1]].set(Wfc)
    bfcp = jnp.full((1, H), -1e30, _F32).at[0, :bfc.shape[0]].set(bfc)

    g = _mm(x, W1, dis16)
    acc = msg(g, row2d, col2d, w3d)
    g = _epi_mm(acc, g, dis16, b1r, W2)
    for _ in range(5):
        acc = msg(g, row2d, col2d, w3d)
        g = _epi_mm(acc, g, dis16, b2r, W2)
    acc = msg(g, row2d, col2d, w3d)
    probs = _final(acc, g, dis16, b2r, wfcp, bfcp)
    return probs[:, :Wfc.shape[1]]


# speedup vs baseline: 3.6232x; 1.0247x over previous
import functools

import jax
import jax.numpy as jnp
from jax import lax
from jax.experimental import pallas as pl
from jax.experimental.pallas import tpu as pltpu
from jax.experimental.pallas import tpu_sc as plsc

N = 10000
NP = 10240
H = 128
LANES = 16
NC, NS = 2, 16
NW = NC * NS
CHUNK = 64
RPT = NP // NS
RQ = RPT // 5

_F32 = jnp.float32
_I32 = jnp.int32


def _mesh():
    return plsc.VectorSubcoreMesh(
        core_axis_name="c", subcore_axis_name="s",
        num_cores=NC, num_subcores=NS)


def _worker_id():
    return lax.axis_index("s") * NC + lax.axis_index("c")


KB = 8
NBUF = 2


def _make_msg_kernel(cpw, nchunk):
    @functools.partial(
        pl.kernel,
        out_type=jax.ShapeDtypeStruct((NC, NP, H), _F32),
        mesh=_mesh(),
        scratch_types=(
            [pltpu.VMEM((KB, CHUNK), _I32),
             pltpu.VMEM((KB, CHUNK), _I32)]
            + [pltpu.VMEM((CHUNK, H), _F32)] * NBUF
            + [pltpu.VMEM((CHUNK, LANES), _F32)] * NBUF
            + [pltpu.VMEM_SHARED((NP, H), _F32),
               pltpu.SemaphoreType.DMA,
               pltpu.SemaphoreType.DMA,
               pltpu.SemaphoreType.DMA]
        ),
    )
    def msg_kernel(g2, row2d, col2d, w3d, out, ridx, cidx, *rest):
        rows = list(rest[:NBUF])
        wv = list(rest[NBUF:2 * NBUF])
        acc, gsem, msem, ssem = rest[2 * NBUF:]
        c = lax.axis_index("c")
        s = lax.axis_index("s")
        wid = _worker_id()

        zbuf = rows[0]

        def zrow(r, _):
            for q in range(H // LANES):
                zbuf[r, pl.ds(q * LANES, LANES)] = jnp.zeros((LANES,), _F32)
            return 0

        lax.fori_loop(0, CHUNK, zrow, 0, unroll=4)
        for k in range(RPT // CHUNK):
            pltpu.sync_copy(zbuf, acc.at[pl.ds(s * RPT + k * CHUNK, CHUNK)])
        plsc.subcore_barrier()

        def scale(b):
            rbuf = rows[b]
            wbuf = wv[b]

            def ebody(e, _):
                wsplat = wbuf[e, :]
                for q in range(H // LANES):
                    sl = pl.ds(q * LANES, LANES)
                    rbuf[e, sl] = rbuf[e, sl] * wsplat
                return 0

            lax.fori_loop(0, CHUNK, ebody, 0, unroll=4)

        def issue(base, jj):
            b = jj % NBUF
            pltpu.async_copy(g2.at[ridx.at[jj]], rows[b], gsem)
            pltpu.async_copy(w3d.at[base + jj], wv[b], msem)

        def drain_gather(b):
            pltpu.make_async_copy(g2.at[ridx.at[0]], rows[b], gsem).wait()
            pltpu.make_async_copy(w3d.at[0], wv[b], msem).wait()

        def drain_scatter(b):
            pltpu.make_async_copy(rows[b], acc.at[cidx.at[0]], ssem).wait()

        def blk(bi, _):
            base = wid * cpw + bi * KB

            @pl.when(bi > 0)
            def _():
                drain_scatter((KB - 1) % NBUF)

            pltpu.sync_copy(row2d.at[pl.ds(base, KB)], ridx)
            pltpu.sync_copy(col2d.at[pl.ds(base, KB)], cidx)

            for jj in range(NBUF - 1):
                issue(base, jj)

            for jj in range(KB):
                b = jj % NBUF
                drain_gather(b)
                scale(b)
                pltpu.async_copy(rows[b], acc.at[cidx.at[jj]], ssem,
                                 add=True)
                if jj > 0:
                    drain_scatter((jj - 1) % NBUF)
                if jj < KB - (NBUF - 1):
                    issue(base, jj + NBUF - 1)
            return 0

        lax.fori_loop(0, cpw // KB, blk, 0)
        drain_scatter((KB - 1) % NBUF)
        plsc.subcore_barrier()
        pltpu.sync_copy(acc.at[pl.ds(s * RPT, RPT)],
                        out.at[c, pl.ds(s * RPT, RPT)])

    return msg_kernel


_BR = 1000


def _dis_body(d0_ref, d1_ref, dis_ref):
    deg = d0_ref[...] + d1_ref[...] + 2.0
    dis_ref[...] = jnp.where(deg > 0.0, lax.rsqrt(deg), 0.0)


def _dis_kernel(d0, d1):
    return pl.pallas_call(
        _dis_body,
        out_shape=jax.ShapeDtypeStruct(d0.shape, _F32),
    )(d0, d1)


def _mm_body(x_ref, w_ref, dis_ref, o_ref):
    o_ref[...] = dis_ref[:, 0:1] * jnp.dot(x_ref[...], w_ref[...],
                                           preferred_element_type=_F32)


def _mm(x, w, dis16):
    n, k = x.shape
    return pl.pallas_call(
        _mm_body,
        grid=(n // _BR,),
        in_specs=[pl.BlockSpec((_BR, k), lambda i: (i, 0)),
                  pl.BlockSpec((k, w.shape[1]), lambda i: (0, 0)),
                  pl.BlockSpec((_BR, H), lambda i: (i, 0))],
        out_specs=pl.BlockSpec((_BR, w.shape[1]), lambda i: (i, 0)),
        out_shape=jax.ShapeDtypeStruct((n, w.shape[1]), _F32),
    )(x, w, dis16)


def _epi_mm_body(acc_ref0, acc_ref1, g_ref, dis_ref, b_ref, w_ref, o_ref):
    dis = dis_ref[:, 0:1]
    act = dis * (acc_ref0[0] + acc_ref1[0] + 2.0 * g_ref[...]) + b_ref[...]
    act = jnp.maximum(act, 0.0)
    o_ref[...] = dis * jnp.dot(act, w_ref[...], preferred_element_type=_F32)


def _epi_mm(acc, g, dis16, b, w):
    return pl.pallas_call(
        _epi_mm_body,
        grid=(N // _BR,),
        in_specs=[pl.BlockSpec((1, _BR, H), lambda i: (0, i, 0)),
                  pl.BlockSpec((1, _BR, H), lambda i: (1, i, 0)),
                  pl.BlockSpec((_BR, H), lambda i: (i, 0)),
                  pl.BlockSpec((_BR, H), lambda i: (i, 0)),
                  pl.BlockSpec((1, H), lambda i: (0, 0)),
                  pl.BlockSpec((H, H), lambda i: (0, 0))],
        out_specs=pl.BlockSpec((_BR, H), lambda i: (i, 0)),
        out_shape=jax.ShapeDtypeStruct((N, H), _F32),
    )(acc, acc, g, dis16, b, w)


def _final_body(acc_ref0, acc_ref1, g_ref, dis_ref, b_ref, wfc_ref, bfc_ref,
                o_ref):
    dis = dis_ref[:, 0:1]
    act = dis * (acc_ref0[0] + acc_ref1[0] + 2.0 * g_ref[...]) + b_ref[...]
    act = jnp.maximum(act, 0.0)
    logits = jnp.dot(act, wfc_ref[...], preferred_element_type=_F32)
    logits = logits + bfc_ref[...]
    m = jnp.max(logits, axis=-1, keepdims=True)
    e = jnp.exp(logits - m)
    o_ref[...] = e / jnp.sum(e, axis=-1, keepdims=True)


def _final(acc, g, dis16, b, wfcp, bfcp):
    return pl.pallas_call(
        _final_body,
        grid=(N // _BR,),
        in_specs=[pl.BlockSpec((1, _BR, H), lambda i: (0, i, 0)),
                  pl.BlockSpec((1, _BR, H), lambda i: (1, i, 0)),
                  pl.BlockSpec((_BR, H), lambda i: (i, 0)),
                  pl.BlockSpec((_BR, H), lambda i: (i, 0)),
                  pl.BlockSpec((1, H), lambda i: (0, 0)),
                  pl.BlockSpec((H, H), lambda i: (0, 0)),
                  pl.BlockSpec((1, H), lambda i: (0, 0))],
        out_specs=pl.BlockSpec((_BR, H), lambda i: (i, 0)),
        out_shape=jax.ShapeDtypeStruct((N, H), _F32),
    )(acc, acc, g, dis16, b, wfcp, bfcp)


def kernel(x, edge_index, edge_attr, W1, b1, W2, b2, Wfc, bfc):
    E = edge_index.shape[1]
    cpw = -(-E // (CHUNK * NW))
    cpw = -(-cpw // KB) * KB
    epad = cpw * CHUNK * NW
    nchunk = epad // CHUNK

    ei = edge_index.astype(_I32)
    row = jnp.pad(ei[0], (0, epad - E))
    pad_cols = jnp.arange(epad - E, dtype=_I32) % NP
    col = jnp.concatenate([ei[1], pad_cols])
    w = jnp.pad(edge_attr.astype(_F32), (0, epad - E))

    row2d = row.reshape(nchunk, CHUNK)
    col2d = col.reshape(nchunk, CHUNK)
    w3d = jnp.broadcast_to(w[:, None], (epad, LANES)).reshape(
        nchunk, CHUNK, LANES)

    msg = _make_msg_kernel(cpw, nchunk)

    deg2 = msg(jnp.ones((N, H), _F32), row2d, col2d, w3d)
    dis16 = _dis_kernel(deg2[0], deg2[1])

    b1r = b1.reshape(1, H)
    b2r = b2.reshape(1, H)
    wfcp = jnp.zeros((H, H), _F32).at[:, :Wfc.shape[1]].set(Wfc)
    bfcp = jnp.full((1, H), -1e30, _F32).at[0, :bfc.shape[0]].set(bfc)

    g = _mm(x, W1, dis16)
    acc = msg(g, row2d, col2d, w3d)
    g = _epi_mm(acc, g, dis16, b1r, W2)
    for _ in range(5):
        acc = msg(g, row2d, col2d, w3d)
        g = _epi_mm(acc, g, dis16, b2r, W2)
    acc = msg(g, row2d, col2d, w3d)
    probs = _final(acc, g, dis16, b2r, wfcp, bfcp)
    return probs[:, :Wfc.shape[1]]


# speedup vs baseline: 4.0034x; 1.1049x over previous
import functools

import jax
import jax.numpy as jnp
from jax import lax
from jax.experimental import pallas as pl
from jax.experimental.pallas import tpu as pltpu
from jax.experimental.pallas import tpu_sc as plsc

N = 10000
NP = 10240
H = 128
LANES = 16
NC, NS = 2, 16
NW = NC * NS
CHUNK = 64
RPT = NP // NS
RQ = RPT // 5

_F32 = jnp.float32
_I32 = jnp.int32


def _mesh():
    return plsc.VectorSubcoreMesh(
        core_axis_name="c", subcore_axis_name="s",
        num_cores=NC, num_subcores=NS)


def _worker_id():
    return lax.axis_index("s") * NC + lax.axis_index("c")


KB = 8
NBUF = 2


def _make_msg_kernel(cpw, nchunk):
    @functools.partial(
        pl.kernel,
        out_type=jax.ShapeDtypeStruct((NC, NP, H), _F32),
        mesh=_mesh(),
        scratch_types=(
            [pltpu.VMEM((KB, CHUNK), _I32),
             pltpu.VMEM((KB, CHUNK), _I32)]
            + [pltpu.VMEM((CHUNK, H), _F32)] * NBUF
            + [pltpu.VMEM((CHUNK, LANES), _F32)] * NBUF
            + [pltpu.VMEM_SHARED((NP, H), _F32),
               pltpu.SemaphoreType.DMA,
               pltpu.SemaphoreType.DMA,
               pltpu.SemaphoreType.DMA]
        ),
    )
    def msg_kernel(g2, row2d, col2d, w3d, out, ridx, cidx, *rest):
        rows = list(rest[:NBUF])
        wv = list(rest[NBUF:2 * NBUF])
        acc, gsem, msem, ssem = rest[2 * NBUF:]
        c = lax.axis_index("c")
        s = lax.axis_index("s")
        wid = _worker_id()

        zbuf = rows[0]

        def zrow(r, _):
            for q in range(H // LANES):
                zbuf[r, pl.ds(q * LANES, LANES)] = jnp.zeros((LANES,), _F32)
            return 0

        lax.fori_loop(0, CHUNK, zrow, 0, unroll=4)
        for k in range(RPT // CHUNK):
            pltpu.sync_copy(zbuf, acc.at[pl.ds(s * RPT + k * CHUNK, CHUNK)])
        plsc.subcore_barrier()

        def scale(b):
            rbuf = rows[b]
            wbuf = wv[b]

            def ebody(e, _):
                wsplat = wbuf[e, :]
                for q in range(H // LANES):
                    sl = pl.ds(q * LANES, LANES)
                    rbuf[e, sl] = rbuf[e, sl] * wsplat
                return 0

            lax.fori_loop(0, CHUNK, ebody, 0, unroll=4)

        def issue(base, jj):
            b = jj % NBUF
            pltpu.async_copy(g2.at[ridx.at[jj]], rows[b], gsem)
            pltpu.async_copy(w3d.at[base + jj], wv[b], msem)

        def drain_gather(b):
            pltpu.make_async_copy(g2.at[ridx.at[0]], rows[b], gsem).wait()
            pltpu.make_async_copy(w3d.at[0], wv[b], msem).wait()

        def drain_scatter(b):
            pltpu.make_async_copy(rows[b], acc.at[cidx.at[0]], ssem).wait()

        def blk(bi, _):
            base = wid * cpw + bi * KB

            @pl.when(bi > 0)
            def _():
                drain_scatter((KB - 1) % NBUF)

            pltpu.sync_copy(row2d.at[pl.ds(base, KB)], ridx)
            pltpu.sync_copy(col2d.at[pl.ds(base, KB)], cidx)

            for jj in range(NBUF - 1):
                issue(base, jj)

            for jj in range(KB):
                b = jj % NBUF
                drain_gather(b)
                scale(b)
                pltpu.async_copy(rows[b], acc.at[cidx.at[jj]], ssem,
                                 add=True)
                if jj > 0:
                    drain_scatter((jj - 1) % NBUF)
                if jj < KB - (NBUF - 1):
                    issue(base, jj + NBUF - 1)
            return 0

        lax.fori_loop(0, cpw // KB, blk, 0)
        drain_scatter((KB - 1) % NBUF)
        plsc.subcore_barrier()
        pltpu.sync_copy(acc.at[pl.ds(s * RPT, RPT)],
                        out.at[c, pl.ds(s * RPT, RPT)])

    return msg_kernel


def _make_deg_kernel(cpw, nchunk):
    @functools.partial(
        pl.kernel,
        out_type=jax.ShapeDtypeStruct((NC, NP, H), _F32),
        mesh=_mesh(),
        scratch_types=(
            [pltpu.VMEM((KB, CHUNK), _I32)]
            + [pltpu.VMEM((CHUNK, H), _F32)] * NBUF
            + [pltpu.VMEM((CHUNK, LANES), _F32)] * NBUF
            + [pltpu.VMEM_SHARED((NP, H), _F32),
               pltpu.SemaphoreType.DMA,
               pltpu.SemaphoreType.DMA]
        ),
    )
    def deg_kernel(col2d, w3d, out, cidx, *rest):
        rows = list(rest[:NBUF])
        wv = list(rest[NBUF:2 * NBUF])
        acc, msem, ssem = rest[2 * NBUF:]
        c = lax.axis_index("c")
        s = lax.axis_index("s")
        wid = _worker_id()

        zbuf = rows[0]

        def zrow(r, _):
            for q in range(H // LANES):
                zbuf[r, pl.ds(q * LANES, LANES)] = jnp.zeros((LANES,), _F32)
            return 0

        lax.fori_loop(0, CHUNK, zrow, 0, unroll=4)
        for k in range(RPT // CHUNK):
            pltpu.sync_copy(zbuf, acc.at[pl.ds(s * RPT + k * CHUNK, CHUNK)])
        plsc.subcore_barrier()

        def widen(b):
            rbuf = rows[b]
            wbuf = wv[b]

            def ebody(e, _):
                wsplat = wbuf[e, :]
                for q in range(H // LANES):
                    rbuf[e, pl.ds(q * LANES, LANES)] = wsplat
                return 0

            lax.fori_loop(0, CHUNK, ebody, 0, unroll=4)

        def drain_scatter(b):
            pltpu.make_async_copy(rows[b], acc.at[cidx.at[0]], ssem).wait()

        def blk(bi, _):
            base = wid * cpw + bi * KB

            @pl.when(bi > 0)
            def _():
                drain_scatter((KB - 1) % NBUF)

            pltpu.sync_copy(col2d.at[pl.ds(base, KB)], cidx)

            for jj in range(NBUF - 1):
                pltpu.async_copy(w3d.at[base + jj], wv[jj % NBUF], msem)

            for jj in range(KB):
                b = jj % NBUF
                pltpu.make_async_copy(w3d.at[0], wv[b], msem).wait()
                widen(b)
                pltpu.async_copy(rows[b], acc.at[cidx.at[jj]], ssem,
                                 add=True)
                if jj > 0:
                    drain_scatter((jj - 1) % NBUF)
                if jj < KB - (NBUF - 1):
                    pltpu.async_copy(w3d.at[base + jj + NBUF - 1],
                                     wv[(jj + NBUF - 1) % NBUF], msem)
            return 0

        lax.fori_loop(0, cpw // KB, blk, 0)
        drain_scatter((KB - 1) % NBUF)
        plsc.subcore_barrier()
        pltpu.sync_copy(acc.at[pl.ds(s * RPT, RPT)],
                        out.at[c, pl.ds(s * RPT, RPT)])

    return deg_kernel


_BR = 1000


def _dis_body(d0_ref, d1_ref, dis_ref):
    deg = d0_ref[...] + d1_ref[...] + 2.0
    dis_ref[...] = jnp.where(deg > 0.0, lax.rsqrt(deg), 0.0)


def _dis_kernel(d0, d1):
    return pl.pallas_call(
        _dis_body,
        out_shape=jax.ShapeDtypeStruct(d0.shape, _F32),
    )(d0, d1)


def _mm_body(x_ref, w_ref, dis_ref, o_ref):
    o_ref[...] = dis_ref[:, 0:1] * jnp.dot(x_ref[...], w_ref[...],
                                           preferred_element_type=_F32)


def _mm(x, w, dis16):
    n, k = x.shape
    return pl.pallas_call(
        _mm_body,
        grid=(n // _BR,),
        in_specs=[pl.BlockSpec((_BR, k), lambda i: (i, 0)),
                  pl.BlockSpec((k, w.shape[1]), lambda i: (0, 0)),
                  pl.BlockSpec((_BR, H), lambda i: (i, 0))],
        out_specs=pl.BlockSpec((_BR, w.shape[1]), lambda i: (i, 0)),
        out_shape=jax.ShapeDtypeStruct((n, w.shape[1]), _F32),
    )(x, w, dis16)


def _epi_mm_body(acc_ref0, acc_ref1, g_ref, dis_ref, b_ref, w_ref, o_ref):
    dis = dis_ref[:, 0:1]
    act = dis * (acc_ref0[0] + acc_ref1[0] + 2.0 * g_ref[...]) + b_ref[...]
    act = jnp.maximum(act, 0.0)
    o_ref[...] = dis * jnp.dot(act, w_ref[...], preferred_element_type=_F32)


def _epi_mm(acc, g, dis16, b, w):
    return pl.pallas_call(
        _epi_mm_body,
        grid=(N // _BR,),
        in_specs=[pl.BlockSpec((1, _BR, H), lambda i: (0, i, 0)),
                  pl.BlockSpec((1, _BR, H), lambda i: (1, i, 0)),
                  pl.BlockSpec((_BR, H), lambda i: (i, 0)),
                  pl.BlockSpec((_BR, H), lambda i: (i, 0)),
                  pl.BlockSpec((1, H), lambda i: (0, 0)),
                  pl.BlockSpec((H, H), lambda i: (0, 0))],
        out_specs=pl.BlockSpec((_BR, H), lambda i: (i, 0)),
        out_shape=jax.ShapeDtypeStruct((N, H), _F32),
    )(acc, acc, g, dis16, b, w)


def _final_body(acc_ref0, acc_ref1, g_ref, dis_ref, b_ref, wfc_ref, bfc_ref,
                o_ref):
    dis = dis_ref[:, 0:1]
    act = dis * (acc_ref0[0] + acc_ref1[0] + 2.0 * g_ref[...]) + b_ref[...]
    act = jnp.maximum(act, 0.0)
    logits = jnp.dot(act, wfc_ref[...], preferred_element_type=_F32)
    logits = logits + bfc_ref[...]
    m = jnp.max(logits, axis=-1, keepdims=True)
    e = jnp.exp(logits - m)
    o_ref[...] = e / jnp.sum(e, axis=-1, keepdims=True)


def _final(acc, g, dis16, b, wfcp, bfcp):
    return pl.pallas_call(
        _final_body,
        grid=(N // _BR,),
        in_specs=[pl.BlockSpec((1, _BR, H), lambda i: (0, i, 0)),
                  pl.BlockSpec((1, _BR, H), lambda i: (1, i, 0)),
                  pl.BlockSpec((_BR, H), lambda i: (i, 0)),
                  pl.BlockSpec((_BR, H), lambda i: (i, 0)),
                  pl.BlockSpec((1, H), lambda i: (0, 0)),
                  pl.BlockSpec((H, H), lambda i: (0, 0)),
                  pl.BlockSpec((1, H), lambda i: (0, 0))],
        out_specs=pl.BlockSpec((_BR, H), lambda i: (i, 0)),
        out_shape=jax.ShapeDtypeStruct((N, H), _F32),
    )(acc, acc, g, dis16, b, wfcp, bfcp)


def kernel(x, edge_index, edge_attr, W1, b1, W2, b2, Wfc, bfc):
    E = edge_index.shape[1]
    cpw = -(-E // (CHUNK * NW))
    cpw = -(-cpw // KB) * KB
    epad = cpw * CHUNK * NW
    nchunk = epad // CHUNK

    ei = edge_index.astype(_I32)
    row = jnp.pad(ei[0], (0, epad - E))
    pad_cols = jnp.arange(epad - E, dtype=_I32) % NP
    col = jnp.concatenate([ei[1], pad_cols])
    w = jnp.pad(edge_attr.astype(_F32), (0, epad - E))

    row2d = row.reshape(nchunk, CHUNK)
    col2d = col.reshape(nchunk, CHUNK)
    w3d = jnp.broadcast_to(w[:, None], (epad, LANES)).reshape(
        nchunk, CHUNK, LANES)

    msg = _make_msg_kernel(cpw, nchunk)

    deg2 = _make_deg_kernel(cpw, nchunk)(col2d, w3d)
    dis16 = _dis_kernel(deg2[0], deg2[1])

    b1r = b1.reshape(1, H)
    b2r = b2.reshape(1, H)
    wfcp = jnp.zeros((H, H), _F32).at[:, :Wfc.shape[1]].set(Wfc)
    bfcp = jnp.full((1, H), -1e30, _F32).at[0, :bfc.shape[0]].set(bfc)

    g = _mm(x, W1, dis16)
    acc = msg(g, row2d, col2d, w3d)
    g = _epi_mm(acc, g, dis16, b1r, W2)
    for _ in range(5):
        acc = msg(g, row2d, col2d, w3d)
        g = _epi_mm(acc, g, dis16, b2r, W2)
    acc = msg(g, row2d, col2d, w3d)
    probs = _final(acc, g, dis16, b2r, wfcp, bfcp)
    return probs[:, :Wfc.shape[1]]


# speedup vs baseline: 4.3193x; 1.0789x over previous
import functools

import jax
import jax.numpy as jnp
from jax import lax
from jax.experimental import pallas as pl
from jax.experimental.pallas import tpu as pltpu
from jax.experimental.pallas import tpu_sc as plsc

N = 10000
NP = 10240
H = 128
LANES = 16
NC, NS = 2, 16
NW = NC * NS
CHUNK = 32
RPT = NP // NS
RQ = RPT // 5

_F32 = jnp.float32
_I32 = jnp.int32


def _mesh():
    return plsc.VectorSubcoreMesh(
        core_axis_name="c", subcore_axis_name="s",
        num_cores=NC, num_subcores=NS)


def _worker_id():
    return lax.axis_index("s") * NC + lax.axis_index("c")


KB = 8
NBUF = 4


def _make_msg_kernel(cpw, nchunk):
    @functools.partial(
        pl.kernel,
        out_type=jax.ShapeDtypeStruct((NC, NP, H), _F32),
        mesh=_mesh(),
        scratch_types=(
            [pltpu.VMEM((KB, CHUNK), _I32),
             pltpu.VMEM((KB, CHUNK), _I32)]
            + [pltpu.VMEM((CHUNK, H), _F32)] * NBUF
            + [pltpu.VMEM((CHUNK, LANES), _F32)] * NBUF
            + [pltpu.VMEM_SHARED((NP, H), _F32),
               pltpu.SemaphoreType.DMA,
               pltpu.SemaphoreType.DMA,
               pltpu.SemaphoreType.DMA]
        ),
    )
    def msg_kernel(g2, row2d, col2d, w3d, out, ridx, cidx, *rest):
        rows = list(rest[:NBUF])
        wv = list(rest[NBUF:2 * NBUF])
        acc, gsem, msem, ssem = rest[2 * NBUF:]
        c = lax.axis_index("c")
        s = lax.axis_index("s")
        wid = _worker_id()

        zbuf = rows[0]

        def zrow(r, _):
            for q in range(H // LANES):
                zbuf[r, pl.ds(q * LANES, LANES)] = jnp.zeros((LANES,), _F32)
            return 0

        lax.fori_loop(0, CHUNK, zrow, 0, unroll=4)
        for k in range(RPT // CHUNK):
            pltpu.sync_copy(zbuf, acc.at[pl.ds(s * RPT + k * CHUNK, CHUNK)])
        plsc.subcore_barrier()

        def scale(b):
            rbuf = rows[b]
            wbuf = wv[b]

            def ebody(e, _):
                wsplat = wbuf[e, :]
                for q in range(H // LANES):
                    sl = pl.ds(q * LANES, LANES)
                    rbuf[e, sl] = rbuf[e, sl] * wsplat
                return 0

            lax.fori_loop(0, CHUNK, ebody, 0, unroll=4)

        def issue(base, jj):
            b = jj % NBUF
            pltpu.async_copy(g2.at[ridx.at[jj]], rows[b], gsem)
            pltpu.async_copy(w3d.at[base + jj], wv[b], msem)

        def drain_gather(b):
            pltpu.make_async_copy(g2.at[ridx.at[0]], rows[b], gsem).wait()
            pltpu.make_async_copy(w3d.at[0], wv[b], msem).wait()

        def drain_scatter(b):
            pltpu.make_async_copy(rows[b], acc.at[cidx.at[0]], ssem).wait()

        def blk(bi, _):
            base = wid * cpw + bi * KB

            @pl.when(bi > 0)
            def _():
                drain_scatter((KB - 1) % NBUF)

            pltpu.sync_copy(row2d.at[pl.ds(base, KB)], ridx)
            pltpu.sync_copy(col2d.at[pl.ds(base, KB)], cidx)

            for jj in range(NBUF - 1):
                issue(base, jj)

            for jj in range(KB):
                b = jj % NBUF
                drain_gather(b)
                scale(b)
                pltpu.async_copy(rows[b], acc.at[cidx.at[jj]], ssem,
                                 add=True)
                if jj > 0:
                    drain_scatter((jj - 1) % NBUF)
                if jj < KB - (NBUF - 1):
                    issue(base, jj + NBUF - 1)
            return 0

        lax.fori_loop(0, cpw // KB, blk, 0)
        drain_scatter((KB - 1) % NBUF)
        plsc.subcore_barrier()
        pltpu.sync_copy(acc.at[pl.ds(s * RPT, RPT)],
                        out.at[c, pl.ds(s * RPT, RPT)])

    return msg_kernel


def _make_deg_kernel(cpw, nchunk):
    @functools.partial(
        pl.kernel,
        out_type=jax.ShapeDtypeStruct((NC, NP, H), _F32),
        mesh=_mesh(),
        scratch_types=(
            [pltpu.VMEM((KB, CHUNK), _I32)]
            + [pltpu.VMEM((CHUNK, H), _F32)] * NBUF
            + [pltpu.VMEM((CHUNK, LANES), _F32)] * NBUF
            + [pltpu.VMEM_SHARED((NP, H), _F32),
               pltpu.SemaphoreType.DMA,
               pltpu.SemaphoreType.DMA]
        ),
    )
    def deg_kernel(col2d, w3d, out, cidx, *rest):
        rows = list(rest[:NBUF])
        wv = list(rest[NBUF:2 * NBUF])
        acc, msem, ssem = rest[2 * NBUF:]
        c = lax.axis_index("c")
        s = lax.axis_index("s")
        wid = _worker_id()

        zbuf = rows[0]

        def zrow(r, _):
            for q in range(H // LANES):
                zbuf[r, pl.ds(q * LANES, LANES)] = jnp.zeros((LANES,), _F32)
            return 0

        lax.fori_loop(0, CHUNK, zrow, 0, unroll=4)
        for k in range(RPT // CHUNK):
            pltpu.sync_copy(zbuf, acc.at[pl.ds(s * RPT + k * CHUNK, CHUNK)])
        plsc.subcore_barrier()

        def widen(b):
            rbuf = rows[b]
            wbuf = wv[b]

            def ebody(e, _):
                wsplat = wbuf[e, :]
                for q in range(H // LANES):
                    rbuf[e, pl.ds(q * LANES, LANES)] = wsplat
                return 0

            lax.fori_loop(0, CHUNK, ebody, 0, unroll=4)

        def drain_scatter(b):
            pltpu.make_async_copy(rows[b], acc.at[cidx.at[0]], ssem).wait()

        def blk(bi, _):
            base = wid * cpw + bi * KB

            @pl.when(bi > 0)
            def _():
                drain_scatter((KB - 1) % NBUF)

            pltpu.sync_copy(col2d.at[pl.ds(base, KB)], cidx)

            for jj in range(NBUF - 1):
                pltpu.async_copy(w3d.at[base + jj], wv[jj % NBUF], msem)

            for jj in range(KB):
                b = jj % NBUF
                pltpu.make_async_copy(w3d.at[0], wv[b], msem).wait()
                widen(b)
                pltpu.async_copy(rows[b], acc.at[cidx.at[jj]], ssem,
                                 add=True)
                if jj > 0:
                    drain_scatter((jj - 1) % NBUF)
                if jj < KB - (NBUF - 1):
                    pltpu.async_copy(w3d.at[base + jj + NBUF - 1],
                                     wv[(jj + NBUF - 1) % NBUF], msem)
            return 0

        lax.fori_loop(0, cpw // KB, blk, 0)
        drain_scatter((KB - 1) % NBUF)
        plsc.subcore_barrier()
        pltpu.sync_copy(acc.at[pl.ds(s * RPT, RPT)],
                        out.at[c, pl.ds(s * RPT, RPT)])

    return deg_kernel


_BR = 1000


def _dis_body(d0_ref, d1_ref, dis_ref):
    deg = d0_ref[...] + d1_ref[...] + 2.0
    dis_ref[...] = jnp.where(deg > 0.0, lax.rsqrt(deg), 0.0)


def _dis_kernel(d0, d1):
    return pl.pallas_call(
        _dis_body,
        out_shape=jax.ShapeDtypeStruct(d0.shape, _F32),
    )(d0, d1)


def _mm_body(x_ref, w_ref, dis_ref, o_ref):
    o_ref[...] = dis_ref[:, 0:1] * jnp.dot(x_ref[...], w_ref[...],
                                           preferred_element_type=_F32)


def _mm(x, w, dis16):
    n, k = x.shape
    return pl.pallas_call(
        _mm_body,
        grid=(n // _BR,),
        in_specs=[pl.BlockSpec((_BR, k), lambda i: (i, 0)),
                  pl.BlockSpec((k, w.shape[1]), lambda i: (0, 0)),
                  pl.BlockSpec((_BR, H), lambda i: (i, 0))],
        out_specs=pl.BlockSpec((_BR, w.shape[1]), lambda i: (i, 0)),
        out_shape=jax.ShapeDtypeStruct((n, w.shape[1]), _F32),
    )(x, w, dis16)


def _epi_mm_body(acc_ref0, acc_ref1, g_ref, dis_ref, b_ref, w_ref, o_ref):
    dis = dis_ref[:, 0:1]
    act = dis * (acc_ref0[0] + acc_ref1[0] + 2.0 * g_ref[...]) + b_ref[...]
    act = jnp.maximum(act, 0.0)
    o_ref[...] = dis * jnp.dot(act, w_ref[...], preferred_element_type=_F32)


def _epi_mm(acc, g, dis16, b, w):
    return pl.pallas_call(
        _epi_mm_body,
        grid=(N // _BR,),
        in_specs=[pl.BlockSpec((1, _BR, H), lambda i: (0, i, 0)),
                  pl.BlockSpec((1, _BR, H), lambda i: (1, i, 0)),
                  pl.BlockSpec((_BR, H), lambda i: (i, 0)),
                  pl.BlockSpec((_BR, H), lambda i: (i, 0)),
                  pl.BlockSpec((1, H), lambda i: (0, 0)),
                  pl.BlockSpec((H, H), lambda i: (0, 0))],
        out_specs=pl.BlockSpec((_BR, H), lambda i: (i, 0)),
        out_shape=jax.ShapeDtypeStruct((N, H), _F32),
    )(acc, acc, g, dis16, b, w)


def _final_body(acc_ref0, acc_ref1, g_ref, dis_ref, b_ref, wfc_ref, bfc_ref,
                o_ref):
    dis = dis_ref[:, 0:1]
    act = dis * (acc_ref0[0] + acc_ref1[0] + 2.0 * g_ref[...]) + b_ref[...]
    act = jnp.maximum(act, 0.0)
    logits = jnp.dot(act, wfc_ref[...], preferred_element_type=_F32)
    logits = logits + bfc_ref[...]
    m = jnp.max(logits, axis=-1, keepdims=True)
    e = jnp.exp(logits - m)
    o_ref[...] = e / jnp.sum(e, axis=-1, keepdims=True)


def _final(acc, g, dis16, b, wfcp, bfcp):
    return pl.pallas_call(
        _final_body,
        grid=(N // _BR,),
        in_specs=[pl.BlockSpec((1, _BR, H), lambda i: (0, i, 0)),
                  pl.BlockSpec((1, _BR, H), lambda i: (1, i, 0)),
                  pl.BlockSpec((_BR, H), lambda i: (i, 0)),
                  pl.BlockSpec((_BR, H), lambda i: (i, 0)),
                  pl.BlockSpec((1, H), lambda i: (0, 0)),
                  pl.BlockSpec((H, H), lambda i: (0, 0)),
                  pl.BlockSpec((1, H), lambda i: (0, 0))],
        out_specs=pl.BlockSpec((_BR, H), lambda i: (i, 0)),
        out_shape=jax.ShapeDtypeStruct((N, H), _F32),
    )(acc, acc, g, dis16, b, wfcp, bfcp)


def kernel(x, edge_index, edge_attr, W1, b1, W2, b2, Wfc, bfc):
    E = edge_index.shape[1]
    cpw = -(-E // (CHUNK * NW))
    cpw = -(-cpw // KB) * KB
    epad = cpw * CHUNK * NW
    nchunk = epad // CHUNK

    ei = edge_index.astype(_I32)
    row = jnp.pad(ei[0], (0, epad - E))
    pad_cols = jnp.arange(epad - E, dtype=_I32) % NP
    col = jnp.concatenate([ei[1], pad_cols])
    w = jnp.pad(edge_attr.astype(_F32), (0, epad - E))

    row2d = row.reshape(nchunk, CHUNK)
    col2d = col.reshape(nchunk, CHUNK)
    w3d = jnp.broadcast_to(w[:, None], (epad, LANES)).reshape(
        nchunk, CHUNK, LANES)

    msg = _make_msg_kernel(cpw, nchunk)

    deg2 = _make_deg_kernel(cpw, nchunk)(col2d, w3d)
    dis16 = _dis_kernel(deg2[0], deg2[1])

    b1r = b1.reshape(1, H)
    b2r = b2.reshape(1, H)
    wfcp = jnp.zeros((H, H), _F32).at[:, :Wfc.shape[1]].set(Wfc)
    bfcp = jnp.full((1, H), -1e30, _F32).at[0, :bfc.shape[0]].set(bfc)

    g = _mm(x, W1, dis16)
    acc = msg(g, row2d, col2d, w3d)
    g = _epi_mm(acc, g, dis16, b1r, W2)
    for _ in range(5):
        acc = msg(g, row2d, col2d, w3d)
        g = _epi_mm(acc, g, dis16, b2r, W2)
    acc = msg(g, row2d, col2d, w3d)
    probs = _final(acc, g, dis16, b2r, wfcp, bfcp)
    return probs[:, :Wfc.shape[1]]


# speedup vs baseline: 5.2513x; 1.2158x over previous
import functools

import jax
import jax.numpy as jnp
from jax import lax
from jax.experimental import pallas as pl
from jax.experimental.pallas import tpu as pltpu
from jax.experimental.pallas import tpu_sc as plsc

N = 10000
NP = 10240
H = 128
LANES = 16
NC, NS = 2, 16
NW = NC * NS
CHUNK = 16
RPT = NP // NS
RQ = RPT // 5

_F32 = jnp.float32
_I32 = jnp.int32


def _mesh():
    return plsc.VectorSubcoreMesh(
        core_axis_name="c", subcore_axis_name="s",
        num_cores=NC, num_subcores=NS)


def _worker_id():
    return lax.axis_index("s") * NC + lax.axis_index("c")


KB = 8
NBUF = 8


def _make_msg_kernel(cpw, nchunk):
    @functools.partial(
        pl.kernel,
        out_type=jax.ShapeDtypeStruct((NC, NP, H), _F32),
        mesh=_mesh(),
        scratch_types=(
            [pltpu.VMEM((KB, CHUNK), _I32),
             pltpu.VMEM((KB, CHUNK), _I32)]
            + [pltpu.VMEM((CHUNK, H), _F32)] * NBUF
            + [pltpu.VMEM((CHUNK, LANES), _F32)] * NBUF
            + [pltpu.VMEM_SHARED((NP, H), _F32),
               pltpu.SemaphoreType.DMA,
               pltpu.SemaphoreType.DMA,
               pltpu.SemaphoreType.DMA]
        ),
    )
    def msg_kernel(g2, row2d, col2d, w3d, out, ridx, cidx, *rest):
        rows = list(rest[:NBUF])
        wv = list(rest[NBUF:2 * NBUF])
        acc, gsem, msem, ssem = rest[2 * NBUF:]
        c = lax.axis_index("c")
        s = lax.axis_index("s")
        wid = _worker_id()

        zbuf = rows[0]

        def zrow(r, _):
            for q in range(H // LANES):
                zbuf[r, pl.ds(q * LANES, LANES)] = jnp.zeros((LANES,), _F32)
            return 0

        lax.fori_loop(0, CHUNK, zrow, 0, unroll=4)
        for k in range(RPT // CHUNK):
            pltpu.sync_copy(zbuf, acc.at[pl.ds(s * RPT + k * CHUNK, CHUNK)])
        plsc.subcore_barrier()

        def scale(b):
            rbuf = rows[b]
            wbuf = wv[b]

            def ebody(e, _):
                wsplat = wbuf[e, :]
                for q in range(H // LANES):
                    sl = pl.ds(q * LANES, LANES)
                    rbuf[e, sl] = rbuf[e, sl] * wsplat
                return 0

            lax.fori_loop(0, CHUNK, ebody, 0, unroll=4)

        def issue(base, jj):
            b = jj % NBUF
            pltpu.async_copy(g2.at[ridx.at[jj]], rows[b], gsem)
            pltpu.async_copy(w3d.at[base + jj], wv[b], msem)

        def drain_gather(b):
            pltpu.make_async_copy(g2.at[ridx.at[0]], rows[b], gsem).wait()
            pltpu.make_async_copy(w3d.at[0], wv[b], msem).wait()

        def drain_scatter(b):
            pltpu.make_async_copy(rows[b], acc.at[cidx.at[0]], ssem).wait()

        def blk(bi, _):
            base = wid * cpw + bi * KB

            @pl.when(bi > 0)
            def _():
                drain_scatter((KB - 1) % NBUF)

            pltpu.sync_copy(row2d.at[pl.ds(base, KB)], ridx)
            pltpu.sync_copy(col2d.at[pl.ds(base, KB)], cidx)

            for jj in range(NBUF - 1):
                issue(base, jj)

            for jj in range(KB):
                b = jj % NBUF
                drain_gather(b)
                scale(b)
                pltpu.async_copy(rows[b], acc.at[cidx.at[jj]], ssem,
                                 add=True)
                if jj > 0:
                    drain_scatter((jj - 1) % NBUF)
                if jj < KB - (NBUF - 1):
                    issue(base, jj + NBUF - 1)
            return 0

        lax.fori_loop(0, cpw // KB, blk, 0)
        drain_scatter((KB - 1) % NBUF)
        plsc.subcore_barrier()
        pltpu.sync_copy(acc.at[pl.ds(s * RPT, RPT)],
                        out.at[c, pl.ds(s * RPT, RPT)])

    return msg_kernel


def _make_deg_kernel(cpw, nchunk):
    @functools.partial(
        pl.kernel,
        out_type=jax.ShapeDtypeStruct((NC, NP, H), _F32),
        mesh=_mesh(),
        scratch_types=(
            [pltpu.VMEM((KB, CHUNK), _I32)]
            + [pltpu.VMEM((CHUNK, H), _F32)] * NBUF
            + [pltpu.VMEM((CHUNK, LANES), _F32)] * NBUF
            + [pltpu.VMEM_SHARED((NP, H), _F32),
               pltpu.SemaphoreType.DMA,
               pltpu.SemaphoreType.DMA]
        ),
    )
    def deg_kernel(col2d, w3d, out, cidx, *rest):
        rows = list(rest[:NBUF])
        wv = list(rest[NBUF:2 * NBUF])
        acc, msem, ssem = rest[2 * NBUF:]
        c = lax.axis_index("c")
        s = lax.axis_index("s")
        wid = _worker_id()

        zbuf = rows[0]

        def zrow(r, _):
            for q in range(H // LANES):
                zbuf[r, pl.ds(q * LANES, LANES)] = jnp.zeros((LANES,), _F32)
            return 0

        lax.fori_loop(0, CHUNK, zrow, 0, unroll=4)
        for k in range(RPT // CHUNK):
            pltpu.sync_copy(zbuf, acc.at[pl.ds(s * RPT + k * CHUNK, CHUNK)])
        plsc.subcore_barrier()

        def widen(b):
            rbuf = rows[b]
            wbuf = wv[b]

            def ebody(e, _):
                wsplat = wbuf[e, :]
                for q in range(H // LANES):
                    rbuf[e, pl.ds(q * LANES, LANES)] = wsplat
                return 0

            lax.fori_loop(0, CHUNK, ebody, 0, unroll=4)

        def drain_scatter(b):
            pltpu.make_async_copy(rows[b], acc.at[cidx.at[0]], ssem).wait()

        def blk(bi, _):
            base = wid * cpw + bi * KB

            @pl.when(bi > 0)
            def _():
                drain_scatter((KB - 1) % NBUF)

            pltpu.sync_copy(col2d.at[pl.ds(base, KB)], cidx)

            for jj in range(NBUF - 1):
                pltpu.async_copy(w3d.at[base + jj], wv[jj % NBUF], msem)

            for jj in range(KB):
                b = jj % NBUF
                pltpu.make_async_copy(w3d.at[0], wv[b], msem).wait()
                widen(b)
                pltpu.async_copy(rows[b], acc.at[cidx.at[jj]], ssem,
                                 add=True)
                if jj > 0:
                    drain_scatter((jj - 1) % NBUF)
                if jj < KB - (NBUF - 1):
                    pltpu.async_copy(w3d.at[base + jj + NBUF - 1],
                                     wv[(jj + NBUF - 1) % NBUF], msem)
            return 0

        lax.fori_loop(0, cpw // KB, blk, 0)
        drain_scatter((KB - 1) % NBUF)
        plsc.subcore_barrier()
        pltpu.sync_copy(acc.at[pl.ds(s * RPT, RPT)],
                        out.at[c, pl.ds(s * RPT, RPT)])

    return deg_kernel


_BR = 1000


def _dis_body(d0_ref, d1_ref, dis_ref):
    deg = d0_ref[...] + d1_ref[...] + 2.0
    dis_ref[...] = jnp.where(deg > 0.0, lax.rsqrt(deg), 0.0)


def _dis_kernel(d0, d1):
    return pl.pallas_call(
        _dis_body,
        out_shape=jax.ShapeDtypeStruct(d0.shape, _F32),
    )(d0, d1)


def _mm_body(x_ref, w_ref, dis_ref, o_ref):
    o_ref[...] = dis_ref[:, 0:1] * jnp.dot(x_ref[...], w_ref[...],
                                           preferred_element_type=_F32)


def _mm(x, w, dis16):
    n, k = x.shape
    return pl.pallas_call(
        _mm_body,
        grid=(n // _BR,),
        in_specs=[pl.BlockSpec((_BR, k), lambda i: (i, 0)),
                  pl.BlockSpec((k, w.shape[1]), lambda i: (0, 0)),
                  pl.BlockSpec((_BR, H), lambda i: (i, 0))],
        out_specs=pl.BlockSpec((_BR, w.shape[1]), lambda i: (i, 0)),
        out_shape=jax.ShapeDtypeStruct((n, w.shape[1]), _F32),
    )(x, w, dis16)


def _epi_mm_body(acc_ref0, acc_ref1, g_ref, dis_ref, b_ref, w_ref, o_ref):
    dis = dis_ref[:, 0:1]
    act = dis * (acc_ref0[0] + acc_ref1[0] + 2.0 * g_ref[...]) + b_ref[...]
    act = jnp.maximum(act, 0.0)
    o_ref[...] = dis * jnp.dot(act, w_ref[...], preferred_element_type=_F32)


def _epi_mm(acc, g, dis16, b, w):
    return pl.pallas_call(
        _epi_mm_body,
        grid=(N // _BR,),
        in_specs=[pl.BlockSpec((1, _BR, H), lambda i: (0, i, 0)),
                  pl.BlockSpec((1, _BR, H), lambda i: (1, i, 0)),
                  pl.BlockSpec((_BR, H), lambda i: (i, 0)),
                  pl.BlockSpec((_BR, H), lambda i: (i, 0)),
                  pl.BlockSpec((1, H), lambda i: (0, 0)),
                  pl.BlockSpec((H, H), lambda i: (0, 0))],
        out_specs=pl.BlockSpec((_BR, H), lambda i: (i, 0)),
        out_shape=jax.ShapeDtypeStruct((N, H), _F32),
    )(acc, acc, g, dis16, b, w)


def _final_body(acc_ref0, acc_ref1, g_ref, dis_ref, b_ref, wfc_ref, bfc_ref,
                o_ref):
    dis = dis_ref[:, 0:1]
    act = dis * (acc_ref0[0] + acc_ref1[0] + 2.0 * g_ref[...]) + b_ref[...]
    act = jnp.maximum(act, 0.0)
    logits = jnp.dot(act, wfc_ref[...], preferred_element_type=_F32)
    logits = logits + bfc_ref[...]
    m = jnp.max(logits, axis=-1, keepdims=True)
    e = jnp.exp(logits - m)
    o_ref[...] = e / jnp.sum(e, axis=-1, keepdims=True)


def _final(acc, g, dis16, b, wfcp, bfcp):
    return pl.pallas_call(
        _final_body,
        grid=(N // _BR,),
        in_specs=[pl.BlockSpec((1, _BR, H), lambda i: (0, i, 0)),
                  pl.BlockSpec((1, _BR, H), lambda i: (1, i, 0)),
                  pl.BlockSpec((_BR, H), lambda i: (i, 0)),
                  pl.BlockSpec((_BR, H), lambda i: (i, 0)),
                  pl.BlockSpec((1, H), lambda i: (0, 0)),
                  pl.BlockSpec((H, H), lambda i: (0, 0)),
                  pl.BlockSpec((1, H), lambda i: (0, 0))],
        out_specs=pl.BlockSpec((_BR, H), lambda i: (i, 0)),
        out_shape=jax.ShapeDtypeStruct((N, H), _F32),
    )(acc, acc, g, dis16, b, wfcp, bfcp)


def kernel(x, edge_index, edge_attr, W1, b1, W2, b2, Wfc, bfc):
    E = edge_index.shape[1]
    cpw = -(-E // (CHUNK * NW))
    cpw = -(-cpw // KB) * KB
    epad = cpw * CHUNK * NW
    nchunk = epad // CHUNK

    ei = edge_index.astype(_I32)
    row = jnp.pad(ei[0], (0, epad - E))
    pad_cols = jnp.arange(epad - E, dtype=_I32) % NP
    col = jnp.concatenate([ei[1], pad_cols])
    w = jnp.pad(edge_attr.astype(_F32), (0, epad - E))

    row2d = row.reshape(nchunk, CHUNK)
    col2d = col.reshape(nchunk, CHUNK)
    w3d = jnp.broadcast_to(w[:, None], (epad, LANES)).reshape(
        nchunk, CHUNK, LANES)

    msg = _make_msg_kernel(cpw, nchunk)

    deg2 = _make_deg_kernel(cpw, nchunk)(col2d, w3d)
    dis16 = _dis_kernel(deg2[0], deg2[1])

    b1r = b1.reshape(1, H)
    b2r = b2.reshape(1, H)
    wfcp = jnp.zeros((H, H), _F32).at[:, :Wfc.shape[1]].set(Wfc)
    bfcp = jnp.full((1, H), -1e30, _F32).at[0, :bfc.shape[0]].set(bfc)

    g = _mm(x, W1, dis16)
    acc = msg(g, row2d, col2d, w3d)
    g = _epi_mm(acc, g, dis16, b1r, W2)
    for _ in range(5):
        acc = msg(g, row2d, col2d, w3d)
        g = _epi_mm(acc, g, dis16, b2r, W2)
    acc = msg(g, row2d, col2d, w3d)
    probs = _final(acc, g, dis16, b2r, wfcp, bfcp)
    return probs[:, :Wfc.shape[1]]
